# baseline (speedup 1.0000x reference)
import functools

import jax
import jax.numpy as jnp
from jax import lax
from jax.experimental import pallas as pl
from jax.experimental.pallas import tpu as pltpu
from jax.experimental.pallas import tpu_sc as plsc


def kernel(seq_in, pos_emb_table):
    N, L, D = seq_in.shape
    P = pos_emb_table.shape[1]
    info = plsc.get_sparse_core_info()
    NC, NS = info.num_cores, info.num_subcores
    NW = NC * NS
    n_per_w = N // NW
    mesh = plsc.VectorSubcoreMesh(core_axis_name="c", subcore_axis_name="s")

    @functools.partial(
        pl.kernel,
        out_type=jax.ShapeDtypeStruct((N, L, D + P), seq_in.dtype),
        mesh=mesh,
        scratch_types=[
            pltpu.VMEM((L, 128), seq_in.dtype),
            pltpu.SemaphoreType.DMA,
        ],
        compiler_params=pltpu.CompilerParams(use_tc_tiling_on_sc=False),
    )
    def _sc(seq_hbm, tab_hbm, out_hbm, out_v, sem):
        wid = lax.axis_index("s") * NC + lax.axis_index("c")
        base = wid * n_per_w
        cp = pltpu.make_async_copy(
            tab_hbm.at[pl.ds(0, L)], out_v.at[:, pl.ds(D, P)], sem)
        cp.start()
        cp.wait()
        for j in range(n_per_w):
            n = base + j
            cp_in = pltpu.make_async_copy(
                seq_hbm.at[n], out_v.at[:, pl.ds(0, D)], sem)
            cp_in.start()
            cp_in.wait()
            cp_out = pltpu.make_async_copy(
                out_v.at[:, pl.ds(0, D + P)], out_hbm.at[n], sem)
            cp_out.start()
            cp_out.wait()

    return _sc(seq_in, pos_emb_table)


# speedup vs baseline: 1.7085x; 1.7085x over previous
import jax
import jax.numpy as jnp
from jax.experimental import pallas as pl
from jax.experimental.pallas import tpu as pltpu

_BN = 16
_K = 8


def _body(seq_hbm, tab_hbm, out_hbm, in_buf, out_buf, pos_buf, in_sem, out_sem, pos_sem):
    nchunks = seq_hbm.shape[0] // _BN
    L = pos_buf.shape[0]
    pos_cp = pltpu.make_async_copy(tab_hbm.at[pl.ds(0, L)], pos_buf, pos_sem)
    pos_cp.start()

    def in_copy(i, slot):
        return pltpu.make_async_copy(
            seq_hbm.at[pl.ds(i * _BN, _BN)], in_buf.at[slot], in_sem.at[slot])

    def out_copy(i, slot):
        return pltpu.make_async_copy(
            out_buf.at[slot], out_hbm.at[pl.ds(i * _BN, _BN)], out_sem.at[slot])

    depth = min(_K, nchunks)
    for i in range(depth):
        in_copy(i, i % _K).start()
    pos_cp.wait()
    pos = pos_buf[...]
    for i in range(nchunks):
        slot = i % _K
        in_copy(i, slot).wait()
        if i >= _K:
            out_copy(i - _K, slot).wait()
        seq = in_buf[slot]
        out_buf[slot] = jnp.concatenate(
            [seq, jnp.broadcast_to(pos[None], (_BN,) + pos.shape)], axis=2)
        out_copy(i, slot).start()
        if i + _K < nchunks:
            in_copy(i + _K, slot).start()
    for i in range(nchunks - depth, nchunks):
        out_copy(i, i % _K).wait()


def kernel(seq_in, pos_emb_table):
    N, L, D = seq_in.shape
    P = pos_emb_table.shape[1]
    return pl.pallas_call(
        _body,
        in_specs=[
            pl.BlockSpec(memory_space=pltpu.MemorySpace.HBM),
            pl.BlockSpec(memory_space=pltpu.MemorySpace.HBM),
        ],
        out_specs=pl.BlockSpec(memory_space=pltpu.MemorySpace.HBM),
        out_shape=jax.ShapeDtypeStruct((N, L, D + P), seq_in.dtype),
        scratch_shapes=[
            pltpu.VMEM((_K, _BN, L, D), seq_in.dtype),
            pltpu.VMEM((_K, _BN, L, D + P), seq_in.dtype),
            pltpu.VMEM((L, P), seq_in.dtype),
            pltpu.SemaphoreType.DMA((_K,)),
            pltpu.SemaphoreType.DMA((_K,)),
            pltpu.SemaphoreType.DMA,
        ],
    )(seq_in, pos_emb_table)


# speedup vs baseline: 1.7199x; 1.0067x over previous
import jax
import jax.numpy as jnp
from jax.experimental import pallas as pl
from jax.experimental.pallas import tpu as pltpu

_BN = 32
_K = 8


def _body(seq_hbm, tab_hbm, out_hbm, in_buf, out_buf, pos_buf, in_sem, out_sem, pos_sem):
    nchunks = seq_hbm.shape[0] // _BN
    L = pos_buf.shape[0]
    pos_cp = pltpu.make_async_copy(tab_hbm.at[pl.ds(0, L)], pos_buf, pos_sem)
    pos_cp.start()

    def in_copy(i, slot):
        return pltpu.make_async_copy(
            seq_hbm.at[pl.ds(i * _BN, _BN)], in_buf.at[slot], in_sem.at[slot])

    def out_copy(i, slot):
        return pltpu.make_async_copy(
            out_buf.at[slot], out_hbm.at[pl.ds(i * _BN, _BN)], out_sem.at[slot])

    depth = min(_K, nchunks)
    for i in range(depth):
        in_copy(i, i % _K).start()
    pos_cp.wait()
    pos = pos_buf[...]
    for i in range(nchunks):
        slot = i % _K
        in_copy(i, slot).wait()
        if i >= _K:
            out_copy(i - _K, slot).wait()
        seq = in_buf[slot]
        out_buf[slot] = jnp.concatenate(
            [seq, jnp.broadcast_to(pos[None], (_BN,) + pos.shape)], axis=2)
        out_copy(i, slot).start()
        if i + _K < nchunks:
            in_copy(i + _K, slot).start()
    for i in range(nchunks - depth, nchunks):
        out_copy(i, i % _K).wait()


def kernel(seq_in, pos_emb_table):
    N, L, D = seq_in.shape
    P = pos_emb_table.shape[1]
    return pl.pallas_call(
        _body,
        in_specs=[
            pl.BlockSpec(memory_space=pltpu.MemorySpace.HBM),
            pl.BlockSpec(memory_space=pltpu.MemorySpace.HBM),
        ],
        out_specs=pl.BlockSpec(memory_space=pltpu.MemorySpace.HBM),
        out_shape=jax.ShapeDtypeStruct((N, L, D + P), seq_in.dtype),
        scratch_shapes=[
            pltpu.VMEM((_K, _BN, L, D), seq_in.dtype),
            pltpu.VMEM((_K, _BN, L, D + P), seq_in.dtype),
            pltpu.VMEM((L, P), seq_in.dtype),
            pltpu.SemaphoreType.DMA((_K,)),
            pltpu.SemaphoreType.DMA((_K,)),
            pltpu.SemaphoreType.DMA,
        ],
    )(seq_in, pos_emb_table)
